# Initial kernel scaffold; baseline (speedup 1.0000x reference)
import jax
import jax.numpy as jnp
from jax.experimental import pallas as pl


def kernel(x, edge_index, W_gcn, b_gcn, W_fuse1, b_fuse1, W_fuse2, b_fuse2):
    raise NotImplementedError("write your pallas kernel here")


# speedup vs baseline: 13.0991x; 13.0991x over previous
import functools

import jax
import jax.numpy as jnp
from jax import lax
from jax.experimental import pallas as pl
from jax.experimental.pallas import tpu as pltpu
from jax.experimental.pallas import tpu_sc as plsc

N_NODES = 10000
N_EDGES = 320000
D_FEAT = 128
N_ANS = 1000

NC = 2
NS = 16
NW = NC * NS
B = 128
CH = 16
NCH = 5
NB = CH * NCH
EPW = NB * B
TOT = NW * EPW
R = N_NODES + 112
RPT = R // NS
RQ = R // B

def _deg_body(dst_hbm, ones_hbm, zeros_hbm, out0, out1, dstv, onesv, dacc,
              sem):
    cid = lax.axis_index("c")
    sid = lax.axis_index("s")
    wid = cid * NS + sid
    pltpu.sync_copy(dst_hbm.at[wid], dstv)
    pltpu.sync_copy(ones_hbm, onesv)

    @pl.when(sid == 0)
    def _():
        pltpu.sync_copy(zeros_hbm, dacc)

    plsc.subcore_barrier()

    def fire(j, carry):
        pltpu.async_copy(onesv, dacc.at[dstv.at[j]], sem, add=True)
        return carry

    lax.fori_loop(0, NB, fire, 0)

    def drain(j, carry):
        pltpu.make_async_copy(onesv, dacc.at[dstv.at[j]], sem).wait()
        return carry

    lax.fori_loop(0, NB, drain, 0)
    plsc.subcore_barrier()

    @pl.when(jnp.logical_and(sid == 0, cid == 0))
    def _():
        pltpu.sync_copy(dacc, out0)

    @pl.when(jnp.logical_and(sid == 0, cid == 1))
    def _():
        pltpu.sync_copy(dacc, out1)


def _agg_body(src_hbm, dst_hbm, xs_hbm, zeros_hbm, out_hbm,
              srcv0, srcv1, dstv0, dstv1, bufa, bufb, acc,
              sema, semb, semsi, semdi):
    cid = lax.axis_index("c")
    sid = lax.axis_index("s")
    wid = cid * NS + sid
    row0 = sid * RPT
    pltpu.sync_copy(zeros_hbm, acc.at[pl.ds(row0, RPT)])
    srcv = (srcv0, srcv1)
    dstv = (dstv0, dstv1)
    pltpu.sync_copy(src_hbm.at[wid, pl.ds(0, CH)], srcv0)
    pltpu.sync_copy(dst_hbm.at[wid, pl.ds(0, CH)], dstv0)
    plsc.subcore_barrier()

    pltpu.async_copy(xs_hbm.at[srcv0.at[0]], bufa, sema)
    for c in range(NCH):
        sv = srcv[c % 2]
        dv = dstv[c % 2]
        if c + 1 < NCH:
            pltpu.async_copy(src_hbm.at[wid, pl.ds((c + 1) * CH, CH)],
                             srcv[(c + 1) % 2], semsi)
            pltpu.async_copy(dst_hbm.at[wid, pl.ds((c + 1) * CH, CH)],
                             dstv[(c + 1) % 2], semdi)

        def body(i, carry, sv=sv, dv=dv):
            j = i * 2
            pltpu.async_copy(xs_hbm.at[sv.at[j + 1]], bufb, semb)
            pltpu.make_async_copy(xs_hbm.at[sv.at[j]], bufa, sema).wait()
            pltpu.sync_copy(bufa, acc.at[dv.at[j]], add=True)
            pltpu.async_copy(xs_hbm.at[sv.at[j + 2]], bufa, sema)
            pltpu.make_async_copy(xs_hbm.at[sv.at[j + 1]], bufb, semb).wait()
            pltpu.sync_copy(bufb, acc.at[dv.at[j + 1]], add=True)
            return carry

        lax.fori_loop(0, CH // 2 - 1, body, 0)
        pltpu.async_copy(xs_hbm.at[sv.at[CH - 1]], bufb, semb)
        pltpu.make_async_copy(xs_hbm.at[sv.at[CH - 2]], bufa, sema).wait()
        pltpu.sync_copy(bufa, acc.at[dv.at[CH - 2]], add=True)
        pltpu.make_async_copy(xs_hbm.at[sv.at[CH - 1]], bufb, semb).wait()
        pltpu.sync_copy(bufb, acc.at[dv.at[CH - 1]], add=True)
        if c + 1 < NCH:
            nsv = srcv[(c + 1) % 2]
            pltpu.make_async_copy(src_hbm.at[wid, pl.ds(0, CH)],
                                  nsv, semsi).wait()
            pltpu.make_async_copy(dst_hbm.at[wid, pl.ds(0, CH)],
                                  dstv[(c + 1) % 2], semdi).wait()
            pltpu.async_copy(xs_hbm.at[nsv.at[0]], bufa, sema)

    plsc.subcore_barrier()
    pltpu.sync_copy(acc.at[pl.ds(row0, RPT)],
                    out_hbm.at[cid, pl.ds(row0, RPT)])


@functools.lru_cache(maxsize=None)
def _sc_kernels():
    mesh = plsc.VectorSubcoreMesh(core_axis_name="c", subcore_axis_name="s",
                                  num_cores=NC, num_subcores=NS)
    deg_k = pl.kernel(
        _deg_body,
        out_type=(jax.ShapeDtypeStruct((R,), jnp.float32),
                  jax.ShapeDtypeStruct((R,), jnp.float32)),
        mesh=mesh,
        scratch_types=[
            pltpu.VMEM((NB, B), jnp.int32),
            pltpu.VMEM((B,), jnp.float32),
            pltpu.VMEM_SHARED((R,), jnp.float32),
            pltpu.SemaphoreType.DMA,
        ],
    )
    agg_k = pl.kernel(
        _agg_body,
        out_type=jax.ShapeDtypeStruct((NC, R, D_FEAT), jnp.float32),
        mesh=mesh,
        scratch_types=[
            pltpu.VMEM((CH, B), jnp.int32),
            pltpu.VMEM((CH, B), jnp.int32),
            pltpu.VMEM((CH, B), jnp.int32),
            pltpu.VMEM((CH, B), jnp.int32),
            pltpu.VMEM((B, D_FEAT), jnp.float32),
            pltpu.VMEM((B, D_FEAT), jnp.float32),
            pltpu.VMEM_SHARED((R, D_FEAT), jnp.float32),
            pltpu.SemaphoreType.DMA,
            pltpu.SemaphoreType.DMA,
            pltpu.SemaphoreType.DMA,
            pltpu.SemaphoreType.DMA,
        ],
    )
    return deg_k, agg_k


def _scale_body(x_ref, d0_ref, d1_ref, o_ref):
    deg = d0_ref[...] + d1_ref[...]
    inv = lax.rsqrt(jnp.maximum(deg, 1.0))
    o_ref[...] = x_ref[...] * inv


def _scale_x(x, d0, d1):
    nblk = 10
    rows = N_NODES // nblk
    return pl.pallas_call(
        _scale_body,
        grid=(nblk,),
        in_specs=[
            pl.BlockSpec((rows, D_FEAT), lambda j: (j, 0)),
            pl.BlockSpec((rows, 1), lambda j: (j, 0)),
            pl.BlockSpec((rows, 1), lambda j: (j, 0)),
        ],
        out_specs=pl.BlockSpec((rows, D_FEAT), lambda j: (j, 0)),
        out_shape=jax.ShapeDtypeStruct((N_NODES, D_FEAT), jnp.float32),
    )(x, d0, d1)


def _final_body(p0_ref, p1_ref, d0_ref, d1_ref, w_ref, bg_ref,
                w1_ref, b1_ref, w2_ref, b2_ref, o_ref, acc_ref, *, nblk):
    j = pl.program_id(0)
    deg = d0_ref[...] + d1_ref[...]
    inv = lax.rsqrt(jnp.maximum(deg, 1.0))
    agg = (p0_ref[...] + p1_ref[...]) * inv
    h = jnp.maximum(jnp.dot(agg, w_ref[...],
                            preferred_element_type=jnp.float32)
                    + bg_ref[...], 0.0)
    s = jnp.sum(h, axis=0, keepdims=True)

    @pl.when(j == 0)
    def _():
        acc_ref[0:1, :] = s

    @pl.when(j > 0)
    def _():
        acc_ref[0:1, :] = acc_ref[0:1, :] + s

    @pl.when(j == nblk - 1)
    def _():
        pooled = acc_ref[0:1, :] * (1.0 / N_NODES)
        z = jnp.dot(pooled, w1_ref[...],
                    preferred_element_type=jnp.float32) + b1_ref[...]
        z = jnp.dot(z, w2_ref[...],
                    preferred_element_type=jnp.float32) + b2_ref[...]
        m = jnp.max(z)
        lse = m + jnp.log(jnp.sum(jnp.exp(z - m)))
        o_ref[...] = z - lse


def _final(p0, p1, d0, d1, W_gcn, b_gcn, W_fuse1, b_fuse1, W_fuse2, b_fuse2):
    nblk = 10
    rows = N_NODES // nblk
    return pl.pallas_call(
        functools.partial(_final_body, nblk=nblk),
        grid=(nblk,),
        in_specs=[
            pl.BlockSpec((rows, D_FEAT), lambda j: (j, 0)),
            pl.BlockSpec((rows, D_FEAT), lambda j: (j, 0)),
            pl.BlockSpec((rows, 1), lambda j: (j, 0)),
            pl.BlockSpec((rows, 1), lambda j: (j, 0)),
            pl.BlockSpec((D_FEAT, D_FEAT), lambda j: (0, 0)),
            pl.BlockSpec((1, D_FEAT), lambda j: (0, 0)),
            pl.BlockSpec((D_FEAT, N_ANS), lambda j: (0, 0)),
            pl.BlockSpec((1, N_ANS), lambda j: (0, 0)),
            pl.BlockSpec((N_ANS, N_ANS), lambda j: (0, 0)),
            pl.BlockSpec((1, N_ANS), lambda j: (0, 0)),
        ],
        out_specs=pl.BlockSpec((1, N_ANS), lambda j: (0, 0)),
        out_shape=jax.ShapeDtypeStruct((1, N_ANS), jnp.float32),
        scratch_shapes=[pltpu.VMEM((8, D_FEAT), jnp.float32)],
    )(p0, p1, d0, d1, W_gcn, b_gcn, W_fuse1, b_fuse1, W_fuse2, b_fuse2)


def kernel(x, edge_index, W_gcn, b_gcn, W_fuse1, b_fuse1, W_fuse2, b_fuse2):
    src = edge_index[0].astype(jnp.int32)
    dst = edge_index[1].astype(jnp.int32)
    pad = TOT - N_EDGES
    srcp = jnp.concatenate([src, jnp.zeros((pad,), jnp.int32)])
    dstp = jnp.concatenate([dst, jnp.full((pad,), N_NODES, jnp.int32)])
    src_b = srcp.reshape(NW, NB, B)
    dst_b = dstp.reshape(NW, NB, B)

    ones_deg = jnp.ones((B,), jnp.float32)
    zeros_deg = jnp.zeros((R,), jnp.float32)
    zeros_agg = jnp.zeros((RPT, D_FEAT), jnp.float32)

    deg_kernel, agg_kernel = _sc_kernels()
    d0g, d1g = deg_kernel(dst_b, ones_deg, zeros_deg)
    d0 = d0g.reshape(R, 1)
    d1 = d1g.reshape(R, 1)

    xs = _scale_x(x, d0, d1)

    aggp = agg_kernel(src_b, dst_b, xs, zeros_agg)

    return _final(aggp[0], aggp[1], d0, d1,
                  W_gcn, b_gcn.reshape(1, D_FEAT),
                  W_fuse1, b_fuse1.reshape(1, N_ANS),
                  W_fuse2, b_fuse2.reshape(1, N_ANS))
